# baseline (speedup 1.0000x reference)
import numpy as np
import jax
import jax.numpy as jnp
from jax.experimental import pallas as pl
from jax.experimental.pallas import tpu as pltpu

_B, _H, _W, _CIN = 8, 224, 224, 96
_PH, _PW = 16, 16
_NPW = 14
_KOUT = 256
_ROWLEN = _W * _CIN
_G = _PW * _CIN

_CR = 32
_NCK = _H // _CR
_NX = _B * _NCK
_NBUF = 8

_CCK = 18816 // 14
_NC = 14
_CBUF = 2

_M2 = (np.arange(_G)[:, None] // _CIN == np.arange(128)[None, :]).astype(np.float32)


def _reduce_kernel(x_hbm, c_hbm, m_ref, o_ref, xbuf, cbuf, s_ref, w_ref,
                   xsem, csem):
    s_ref[...] = jnp.zeros_like(s_ref)
    w_ref[...] = jnp.zeros_like(w_ref)

    def x_copy(k, slot):
        b = k // _NCK
        r = (k % _NCK) * _CR
        return pltpu.make_async_copy(
            x_hbm.at[b, pl.ds(r, _CR)], xbuf.at[slot], xsem.at[slot])

    def c_copy(k, slot):
        return pltpu.make_async_copy(
            c_hbm.at[pl.ds(k * _CCK, _CCK)], cbuf.at[slot], csem.at[slot])

    for k in range(_NBUF):
        x_copy(k, k).start()
    for k in range(_CBUF):
        c_copy(k, k).start()

    def c_body(k, carry):
        slot = k % _CBUF
        c_copy(k, slot).wait()
        cr = cbuf.at[slot]
        w = jnp.zeros((1, _KOUT), jnp.float32)
        for t in range(6):
            w = w + cr[pl.ds(224 * t, 224), :].sum(axis=0, keepdims=True)

        @pl.when(k + _CBUF < _NC)
        def _next():
            c_copy(k + _CBUF, slot).start()

        w_ref[...] += w
        return carry

    def x_body(k, carry):
        slot = k % _NBUF
        x_copy(k, slot).wait()
        xr = xbuf.at[slot]
        acc = xr[:, 0:_G]
        for j in range(1, _NPW):
            acc = acc + xr[:, pl.ds(_G * j, _G)]
        acc = acc[0:_PH, :] + acc[_PH:_CR, :]
        sb = jnp.dot(acc, m_ref[...],
                     preferred_element_type=jnp.float32)

        @pl.when(k + _NBUF < _NX)
        def _next():
            x_copy(k + _NBUF, slot).start()

        b = k // _NCK
        s_ref[pl.ds(_PH * b, _PH), :] += sb
        return carry

    jax.lax.fori_loop(0, _NC, c_body, 0, unroll=False)
    jax.lax.fori_loop(0, _NX, x_body, 0, unroll=False)
    for t in range(_B):
        o_ref[pl.ds(_PH * t, _PH)] = (
            s_ref[pl.ds(_PH * t, _PH), 0:_PW].reshape(_PH, _PW, 1)
            * w_ref[...].reshape(1, 1, _KOUT))


def kernel(x, C):
    x2 = x.reshape(_B, _H, _ROWLEN)
    c2 = C.reshape(18816, _KOUT)
    m2 = jnp.asarray(_M2)
    out3 = pl.pallas_call(
        _reduce_kernel,
        in_specs=[
            pl.BlockSpec(memory_space=pltpu.HBM),
            pl.BlockSpec(memory_space=pltpu.HBM),
            pl.BlockSpec(memory_space=pltpu.MemorySpace.VMEM),
        ],
        out_specs=pl.BlockSpec(memory_space=pltpu.MemorySpace.VMEM),
        out_shape=jax.ShapeDtypeStruct((_B * _PH, _PW, _KOUT), jnp.float32),
        scratch_shapes=[
            pltpu.VMEM((_NBUF, _CR, _ROWLEN), jnp.float32),
            pltpu.VMEM((_CBUF, _CCK, _KOUT), jnp.float32),
            pltpu.VMEM((_B * _PH, 128), jnp.float32),
            pltpu.VMEM((1, _KOUT), jnp.float32),
            pltpu.SemaphoreType.DMA((_NBUF,)),
            pltpu.SemaphoreType.DMA((_CBUF,)),
        ],
    )(x2, c2, m2)
    out = out3.reshape(_B, _PH, _PW, _KOUT).transpose(0, 3, 1, 2)
    return out.reshape(_B, _KOUT, 1, _PH, _PW)


# speedup vs baseline: 3.7906x; 3.7906x over previous
import jax
import jax.numpy as jnp
from jax.experimental import pallas as pl
from jax.experimental.pallas import tpu as pltpu

_B, _H, _W, _CIN = 8, 224, 224, 96
_PH, _PW = 16, 16
_NPW = 14
_KOUT = 256
_M = _B * _PH * _PW
_MB = _PH * _PW

_CR = 32
_NCK = _H // _CR
_NX = _B * _NCK
_NBUF = 8

_CCK = 18816 // 14
_NC = 14
_CBUF = 2


def _reduce_kernel(x_hbm, c_hbm, o_ref, *bufs):
    xbufs = bufs[:_NBUF]
    cbufs = bufs[_NBUF:_NBUF + _CBUF]
    s_ref, w_ref = bufs[_NBUF + _CBUF:_NBUF + _CBUF + 2]
    xsems = bufs[_NBUF + _CBUF + 2:_NBUF + _CBUF + 2 + _NBUF]
    csems = bufs[_NBUF + _CBUF + 2 + _NBUF:]

    s_ref[...] = jnp.zeros_like(s_ref)
    w_ref[...] = jnp.zeros_like(w_ref)

    def x_copy(k):
        b, r = k // _NCK, (k % _NCK) * _CR
        return pltpu.make_async_copy(
            x_hbm.at[b, pl.ds(r, _CR)], xbufs[k % _NBUF], xsems[k % _NBUF])

    def c_copy(k):
        return pltpu.make_async_copy(
            c_hbm.at[pl.ds(k * _CCK, _CCK)], cbufs[k % _CBUF], csems[k % _CBUF])

    for k in range(_NBUF):
        x_copy(k).start()
    for k in range(_CBUF):
        c_copy(k).start()

    for k in range(_NC):
        c_copy(k).wait()
        cr = cbufs[k % _CBUF]
        w = jnp.zeros((1, _KOUT), jnp.float32)
        for t in range(6):
            w = w + cr[pl.ds(224 * t, 224), :].sum(axis=0, keepdims=True)
        if k + _CBUF < _NC:
            c_copy(k + _CBUF).start()
        w_ref[...] += w

    for k in range(_NX):
        x_copy(k).wait()
        xr = xbufs[k % _NBUF]
        acc = xr[:, 0:_PW, :]
        for j in range(1, _NPW):
            acc = acc + xr[:, _PW * j:_PW * (j + 1), :]
        acc = acc[0:_PH] + acc[_PH:_CR]
        part = acc.reshape(_MB, _CIN).sum(axis=1, keepdims=True)
        if k + _NBUF < _NX:
            x_copy(k + _NBUF).start()
        s_ref[pl.ds(_MB * (k // _NCK), _MB), :] += part

    for t in range(_B):
        o_ref[pl.ds(_MB * t, _MB), :] = (
            s_ref[pl.ds(_MB * t, _MB), :] * w_ref[...])


def kernel(x, C):
    c2 = C.reshape(18816, _KOUT)
    out2 = pl.pallas_call(
        _reduce_kernel,
        in_specs=[
            pl.BlockSpec(memory_space=pltpu.HBM),
            pl.BlockSpec(memory_space=pltpu.HBM),
        ],
        out_specs=pl.BlockSpec(memory_space=pltpu.MemorySpace.VMEM),
        out_shape=jax.ShapeDtypeStruct((_M, _KOUT), jnp.float32),
        scratch_shapes=(
            [pltpu.VMEM((_CR, _W, _CIN), jnp.float32)] * _NBUF
            + [pltpu.VMEM((_CCK, _KOUT), jnp.float32)] * _CBUF
            + [pltpu.VMEM((_M, 1), jnp.float32),
               pltpu.VMEM((1, _KOUT), jnp.float32)]
            + [pltpu.SemaphoreType.DMA] * (_NBUF + _CBUF)
        ),
    )(x, c2)
    out = out2.reshape(_B, _PH, _PW, _KOUT).transpose(0, 3, 1, 2)
    return out.reshape(_B, _KOUT, 1, _PH, _PW)


# speedup vs baseline: 14.2473x; 3.7586x over previous
import numpy as np
import jax
import jax.numpy as jnp
from jax.experimental import pallas as pl
from jax.experimental.pallas import tpu as pltpu

_B, _H, _W, _CIN = 8, 224, 224, 96
_PH, _PW = 16, 16
_KOUT = 256

_CR = 32
_NCK = _H // _CR
_NX = _B * _NCK
_NBUF = 8
_XR = _CR * _CIN

_CROWS = 2 * 18816
_CCK = _CROWS // 14
_NC = 14
_CBUF = 2

_MW = (np.arange(_W)[:, None] % _PW == np.arange(128)[None, :]).astype(np.float32)
_P2 = ((np.arange(_XR)[None, :] // _CIN) % _PH
       == np.arange(_PH)[:, None]).astype(np.float32)
_PC = (np.arange(_CCK)[None, :] % 2 == np.arange(2)[:, None]).astype(np.float32)


def _reduce_kernel(x_hbm, c_hbm, mw_ref, p2_ref, pc_ref, o_ref,
                   xbuf, cbuf, s_ref, w_ref, xsem, csem):
    s_ref[...] = jnp.zeros_like(s_ref)
    w_ref[...] = jnp.zeros_like(w_ref)

    def x_copy(k, slot):
        b, r = k // _NCK, (k % _NCK) * _CR
        return pltpu.make_async_copy(
            x_hbm.at[b, pl.ds(r, _CR)], xbuf.at[slot], xsem.at[slot])

    def c_copy(k, slot):
        return pltpu.make_async_copy(
            c_hbm.at[pl.ds(k * _CCK, _CCK)], cbuf.at[slot], csem.at[slot])

    for k in range(_NBUF):
        x_copy(k, k).start()
    for k in range(_CBUF):
        c_copy(k, k).start()

    def c_body(k, carry):
        slot = k % _CBUF
        c_copy(k, slot).wait()
        wc = jnp.dot(pc_ref[...], cbuf[slot],
                     preferred_element_type=jnp.float32)

        @pl.when(k + _CBUF < _NC)
        def _next():
            c_copy(k + _CBUF, slot).start()

        w_ref[...] += wc
        return carry

    def x_body(k, carry):
        slot = k % _NBUF
        x_copy(k, slot).wait()
        x2 = xbuf[slot].reshape(_XR, _W).astype(jnp.bfloat16)
        t1 = jnp.dot(x2, mw_ref[...],
                     preferred_element_type=jnp.float32)
        sb = jnp.dot(p2_ref[...], t1,
                     preferred_element_type=jnp.float32)

        @pl.when(k + _NBUF < _NX)
        def _next():
            x_copy(k + _NBUF, slot).start()

        b = k // _NCK
        s_ref[pl.ds(_PH * b, _PH), :] += sb
        return carry

    jax.lax.fori_loop(0, _NC, c_body, 0, unroll=False)
    jax.lax.fori_loop(0, _NX, x_body, 0, unroll=False)

    w = w_ref[...].reshape(1, 1, _KOUT)
    for t in range(_B):
        o_ref[pl.ds(_PH * t, _PH)] = (
            s_ref[pl.ds(_PH * t, _PH), 0:_PW].reshape(_PH, _PW, 1) * w)


def kernel(x, C):
    xt = jnp.transpose(x, (0, 1, 3, 2))
    c3 = C.reshape(_CROWS, 128)
    mw = jnp.asarray(_MW, dtype=jnp.bfloat16)
    p2 = jnp.asarray(_P2)
    pc = jnp.asarray(_PC)
    out3 = pl.pallas_call(
        _reduce_kernel,
        in_specs=[
            pl.BlockSpec(memory_space=pltpu.HBM),
            pl.BlockSpec(memory_space=pltpu.HBM),
            pl.BlockSpec(memory_space=pltpu.MemorySpace.VMEM),
            pl.BlockSpec(memory_space=pltpu.MemorySpace.VMEM),
            pl.BlockSpec(memory_space=pltpu.MemorySpace.VMEM),
        ],
        out_specs=pl.BlockSpec(memory_space=pltpu.MemorySpace.VMEM),
        out_shape=jax.ShapeDtypeStruct((_B * _PH, _PW, _KOUT), jnp.float32),
        scratch_shapes=[
            pltpu.VMEM((_NBUF, _CR, _CIN, _W), jnp.float32),
            pltpu.VMEM((_CBUF, _CCK, 128), jnp.float32),
            pltpu.VMEM((_B * _PH, 128), jnp.float32),
            pltpu.VMEM((2, 128), jnp.float32),
            pltpu.SemaphoreType.DMA((_NBUF,)),
            pltpu.SemaphoreType.DMA((_CBUF,)),
        ],
    )(xt, c3, mw, p2, pc)
    out = out3.reshape(_B, _PH, _PW, _KOUT).transpose(0, 3, 1, 2)
    return out.reshape(_B, _KOUT, 1, _PH, _PW)


# speedup vs baseline: 14.8431x; 1.0418x over previous
import numpy as np
import jax
import jax.numpy as jnp
from jax.experimental import pallas as pl
from jax.experimental.pallas import tpu as pltpu

_B, _H, _W, _CIN = 8, 224, 224, 96
_PH, _PW = 16, 16
_KOUT = 256

_CR = 32
_NCK = _H // _CR
_NX = _B * _NCK
_NBUF = 8
_XR = _CR * _CIN

_CROWS = 2 * 18816
_CCK = _CROWS // 14
_NC = 14
_CBUF = 2

_MW = (np.arange(_W)[:, None] % _PW == np.arange(128)[None, :]).astype(np.float32)
_P2 = ((np.arange(_XR)[None, :] // _CIN) % _PH
       == np.arange(_PH)[:, None]).astype(np.float32)
_PC = (np.arange(_CCK)[None, :] % 2 == np.arange(2)[:, None]).astype(np.float32)


def _reduce_kernel(x_hbm, c_hbm, mw_ref, p2_ref, pc_ref, o_ref,
                   xbuf, cbuf, s_ref, w_ref, xsem, csem):
    s_ref[...] = jnp.zeros_like(s_ref)
    w_ref[...] = jnp.zeros_like(w_ref)

    def x_copy(k, slot):
        b, r = k // _NCK, (k % _NCK) * _CR
        return pltpu.make_async_copy(
            x_hbm.at[b, pl.ds(r, _CR)], xbuf.at[slot], xsem.at[slot])

    def c_copy(k, slot):
        return pltpu.make_async_copy(
            c_hbm.at[pl.ds(k * _CCK, _CCK)], cbuf.at[slot], csem.at[slot])

    for k in range(_NBUF):
        x_copy(k, k).start()
    for k in range(_CBUF):
        c_copy(k, k).start()

    def c_body(k, carry):
        slot = k % _CBUF
        c_copy(k, slot).wait()
        wc = jnp.dot(pc_ref[...], cbuf[slot],
                     preferred_element_type=jnp.float32)

        @pl.when(k + _CBUF < _NC)
        def _next():
            c_copy(k + _CBUF, slot).start()

        w_ref[...] += wc
        return carry

    def x_body(k, carry):
        slot = k % _NBUF
        x_copy(k, slot).wait()
        x2 = xbuf[slot].reshape(_XR, _W)
        t1 = jnp.dot(p2_ref[...], x2,
                     preferred_element_type=jnp.float32)
        sb = jnp.dot(t1, mw_ref[...],
                     preferred_element_type=jnp.float32)

        @pl.when(k + _NBUF < _NX)
        def _next():
            x_copy(k + _NBUF, slot).start()

        b = k // _NCK
        s_ref[pl.ds(_PH * b, _PH), :] += sb
        return carry

    jax.lax.fori_loop(0, _NC, c_body, 0, unroll=False)
    jax.lax.fori_loop(0, _NX, x_body, 0, unroll=False)

    w = w_ref[...].reshape(1, 1, _KOUT)
    for t in range(_B):
        o_ref[pl.ds(_PH * t, _PH)] = (
            s_ref[pl.ds(_PH * t, _PH), 0:_PW].reshape(_PH, _PW, 1) * w)


def kernel(x, C):
    xt = jnp.transpose(x, (0, 1, 3, 2))
    c3 = C.reshape(_CROWS, 128)
    mw = jnp.asarray(_MW)
    p2 = jnp.asarray(_P2)
    pc = jnp.asarray(_PC)
    out3 = pl.pallas_call(
        _reduce_kernel,
        in_specs=[
            pl.BlockSpec(memory_space=pltpu.HBM),
            pl.BlockSpec(memory_space=pltpu.HBM),
            pl.BlockSpec(memory_space=pltpu.MemorySpace.VMEM),
            pl.BlockSpec(memory_space=pltpu.MemorySpace.VMEM),
            pl.BlockSpec(memory_space=pltpu.MemorySpace.VMEM),
        ],
        out_specs=pl.BlockSpec(memory_space=pltpu.MemorySpace.VMEM),
        out_shape=jax.ShapeDtypeStruct((_B * _PH, _PW, _KOUT), jnp.float32),
        scratch_shapes=[
            pltpu.VMEM((_NBUF, _CR, _CIN, _W), jnp.float32),
            pltpu.VMEM((_CBUF, _CCK, 128), jnp.float32),
            pltpu.VMEM((_B * _PH, 128), jnp.float32),
            pltpu.VMEM((2, 128), jnp.float32),
            pltpu.SemaphoreType.DMA((_NBUF,)),
            pltpu.SemaphoreType.DMA((_CBUF,)),
        ],
    )(xt, c3, mw, p2, pc)
    out = out3.reshape(_B, _PH, _PW, _KOUT).transpose(0, 3, 1, 2)
    return out.reshape(_B, _KOUT, 1, _PH, _PW)


# speedup vs baseline: 15.3727x; 1.0357x over previous
import numpy as np
import jax
import jax.numpy as jnp
from jax.experimental import pallas as pl
from jax.experimental.pallas import tpu as pltpu

_B, _H, _W, _CIN = 8, 224, 224, 96
_PH, _PW = 16, 16
_KOUT = 256

_CR = 112
_NCK = _H // _CR
_NX = _B * _NCK
_NBUF = 4
_XR = _CR * _CIN

_CROWS = 2 * 18816
_CCK = _CROWS // 14
_NC = 14
_CBUF = 2

_P2 = ((np.arange(_XR)[None, :] // _CIN) % _PH
       == np.arange(_PH)[:, None]).astype(np.float32)
_MW = (np.arange(_W)[:, None] % _PW == np.arange(128)[None, :]).astype(np.float32)
_PC = (np.arange(_CCK)[None, :] % 2 == np.arange(2)[:, None]).astype(np.float32)


def _reduce_kernel(x_hbm, c_hbm, mw_ref, p2_ref, pc_ref, o_ref,
                   xbuf, cbuf, s_ref, w_ref, xsem, csem):
    s_ref[...] = jnp.zeros_like(s_ref)
    w_ref[...] = jnp.zeros_like(w_ref)

    def x_copy(k, slot):
        b, r = k // _NCK, (k % _NCK) * _CR
        return pltpu.make_async_copy(
            x_hbm.at[b, pl.ds(r, _CR)], xbuf.at[slot], xsem.at[slot])

    def c_copy(k, slot):
        return pltpu.make_async_copy(
            c_hbm.at[pl.ds(k * _CCK, _CCK)], cbuf.at[slot], csem.at[slot])

    for k in range(_NBUF):
        x_copy(k, k).start()
    for k in range(_CBUF):
        c_copy(k, k).start()

    def x_body(k, carry):
        slot = k % _NBUF
        x_copy(k, slot).wait()
        x2 = xbuf[slot].reshape(_XR, _W)
        t1 = jnp.dot(p2_ref[...], x2,
                     preferred_element_type=jnp.float32)
        sb = jnp.dot(t1, mw_ref[...],
                     preferred_element_type=jnp.float32)

        @pl.when(k + _NBUF < _NX)
        def _next():
            x_copy(k + _NBUF, slot).start()

        b = k // _NCK
        s_ref[pl.ds(_PH * b, _PH), :] += sb

        @pl.when(k < _NC)
        def _cstep():
            cslot = k % _CBUF
            c_copy(k, cslot).wait()
            wc = jnp.dot(pc_ref[...], cbuf[cslot],
                         preferred_element_type=jnp.float32)

            @pl.when(k + _CBUF < _NC)
            def _cnext():
                c_copy(k + _CBUF, cslot).start()

            w_ref[...] += wc

        return carry

    jax.lax.fori_loop(0, _NX, x_body, 0, unroll=False)

    w = w_ref[...].reshape(1, 1, _KOUT)
    for t in range(_B):
        o_ref[pl.ds(_PH * t, _PH)] = (
            s_ref[pl.ds(_PH * t, _PH), 0:_PW].reshape(_PH, _PW, 1) * w)


def kernel(x, C):
    xt = jnp.transpose(x, (0, 1, 3, 2))
    c3 = C.reshape(_CROWS, 128)
    mw = jnp.asarray(_MW)
    p2 = jnp.asarray(_P2)
    pc = jnp.asarray(_PC)
    out3 = pl.pallas_call(
        _reduce_kernel,
        in_specs=[
            pl.BlockSpec(memory_space=pltpu.HBM),
            pl.BlockSpec(memory_space=pltpu.HBM),
            pl.BlockSpec(memory_space=pltpu.MemorySpace.VMEM),
            pl.BlockSpec(memory_space=pltpu.MemorySpace.VMEM),
            pl.BlockSpec(memory_space=pltpu.MemorySpace.VMEM),
        ],
        out_specs=pl.BlockSpec(memory_space=pltpu.MemorySpace.VMEM),
        out_shape=jax.ShapeDtypeStruct((_B * _PH, _PW, _KOUT), jnp.float32),
        scratch_shapes=[
            pltpu.VMEM((_NBUF, _CR, _CIN, _W), jnp.float32),
            pltpu.VMEM((_CBUF, _CCK, 128), jnp.float32),
            pltpu.VMEM((_B * _PH, 128), jnp.float32),
            pltpu.VMEM((2, 128), jnp.float32),
            pltpu.SemaphoreType.DMA((_NBUF,)),
            pltpu.SemaphoreType.DMA((_CBUF,)),
        ],
    )(xt, c3, mw, p2, pc)
    out = out3.reshape(_B, _PH, _PW, _KOUT).transpose(0, 3, 1, 2)
    return out.reshape(_B, _KOUT, 1, _PH, _PW)


# speedup vs baseline: 16.5079x; 1.0738x over previous
import numpy as np
import jax
import jax.numpy as jnp
from jax.experimental import pallas as pl
from jax.experimental.pallas import tpu as pltpu

_B, _H, _W, _CIN = 8, 224, 224, 96
_PH, _PW = 16, 16
_KOUT = 256

_CR = 56
_NCK = _H // _CR
_NX = _B * _NCK
_NBUF = 8
_XR = _CR * _CIN

_CROWS = 2 * 18816
_CCK = _CROWS // 14
_NC = 14
_CBUF = 2

_P2 = np.stack([
    (((np.arange(_XR)[None, :] // _CIN) + 8 * p) % _PH
     == np.arange(_PH)[:, None]) for p in range(2)]).astype(np.float32)
_MW = (np.arange(_W)[:, None] % _PW == np.arange(128)[None, :]).astype(np.float32)
_PC = (np.arange(_CCK)[None, :] % 2 == np.arange(2)[:, None]).astype(np.float32)


def _reduce_kernel(x_hbm, c_hbm, mw_ref, p2_ref, pc_ref, o_ref,
                   xbuf, cbuf, s_ref, w_ref, xsem, csem):
    s_ref[...] = jnp.zeros_like(s_ref)
    w_ref[...] = jnp.zeros_like(w_ref)

    def x_copy(k, slot):
        b, r = k // _NCK, (k % _NCK) * _CR
        return pltpu.make_async_copy(
            x_hbm.at[b, pl.ds(r, _CR)], xbuf.at[slot], xsem.at[slot])

    def c_copy(k, slot):
        return pltpu.make_async_copy(
            c_hbm.at[pl.ds(k * _CCK, _CCK)], cbuf.at[slot], csem.at[slot])

    for k in range(_NBUF):
        x_copy(k, k).start()
    for k in range(_CBUF):
        c_copy(k, k).start()

    def x_body(k, carry):
        slot = k % _NBUF
        x_copy(k, slot).wait()
        x2 = xbuf[slot].reshape(_XR, _W)
        t1 = jnp.dot(p2_ref[k % 2], x2,
                     preferred_element_type=jnp.float32)
        sb = jnp.dot(t1, mw_ref[...],
                     preferred_element_type=jnp.float32)

        @pl.when(k + _NBUF < _NX)
        def _next():
            x_copy(k + _NBUF, slot).start()

        b = k // _NCK
        s_ref[pl.ds(_PH * b, _PH), :] += sb

        @pl.when(k < _NC)
        def _cstep():
            cslot = k % _CBUF
            c_copy(k, cslot).wait()
            wc = jnp.dot(pc_ref[...], cbuf[cslot],
                         preferred_element_type=jnp.float32)

            @pl.when(k + _CBUF < _NC)
            def _cnext():
                c_copy(k + _CBUF, cslot).start()

            w_ref[...] += wc

        return carry

    jax.lax.fori_loop(0, _NX, x_body, 0, unroll=False)

    w = w_ref[...].reshape(1, 1, _KOUT)
    for t in range(_B):
        o_ref[pl.ds(_PH * t, _PH)] = (
            s_ref[pl.ds(_PH * t, _PH), 0:_PW].reshape(_PH, _PW, 1) * w)


def kernel(x, C):
    xt = jnp.transpose(x, (0, 1, 3, 2))
    c3 = C.reshape(_CROWS, 128)
    mw = jnp.asarray(_MW)
    p2 = jnp.asarray(_P2)
    pc = jnp.asarray(_PC)
    out3 = pl.pallas_call(
        _reduce_kernel,
        in_specs=[
            pl.BlockSpec(memory_space=pltpu.HBM),
            pl.BlockSpec(memory_space=pltpu.HBM),
            pl.BlockSpec(memory_space=pltpu.MemorySpace.VMEM),
            pl.BlockSpec(memory_space=pltpu.MemorySpace.VMEM),
            pl.BlockSpec(memory_space=pltpu.MemorySpace.VMEM),
        ],
        out_specs=pl.BlockSpec(memory_space=pltpu.MemorySpace.VMEM),
        out_shape=jax.ShapeDtypeStruct((_B * _PH, _PW, _KOUT), jnp.float32),
        scratch_shapes=[
            pltpu.VMEM((_NBUF, _CR, _CIN, _W), jnp.float32),
            pltpu.VMEM((_CBUF, _CCK, 128), jnp.float32),
            pltpu.VMEM((_B * _PH, 128), jnp.float32),
            pltpu.VMEM((2, 128), jnp.float32),
            pltpu.SemaphoreType.DMA((_NBUF,)),
            pltpu.SemaphoreType.DMA((_CBUF,)),
        ],
    )(xt, c3, mw, p2, pc)
    out = out3.reshape(_B, _PH, _PW, _KOUT).transpose(0, 3, 1, 2)
    return out.reshape(_B, _KOUT, 1, _PH, _PW)


# speedup vs baseline: 16.5429x; 1.0021x over previous
import numpy as np
import jax
import jax.numpy as jnp
from jax.experimental import pallas as pl
from jax.experimental.pallas import tpu as pltpu

_B, _H, _W, _CIN = 8, 224, 224, 96
_PH, _PW = 16, 16
_KOUT = 256

_CR = 28
_NCK = _H // _CR
_NX = _B * _NCK
_NBUF = 16
_XR = _CR * _CIN

_CROWS = 2 * 18816
_CCK = _CROWS // 14
_NC = 14
_CBUF = 2

_P2 = np.stack([
    (((np.arange(_XR)[None, :] // _CIN) + 12 * p) % _PH
     == np.arange(_PH)[:, None]) for p in range(4)]).astype(np.float32)
_MW = (np.arange(_W)[:, None] % _PW == np.arange(128)[None, :]).astype(np.float32)
_PC = (np.arange(_CCK)[None, :] % 2 == np.arange(2)[:, None]).astype(np.float32)


def _reduce_kernel(x_hbm, c_hbm, mw_ref, p2_ref, pc_ref, o_ref,
                   xbuf, cbuf, s_ref, w_ref, xsem, csem):
    s_ref[...] = jnp.zeros_like(s_ref)
    w_ref[...] = jnp.zeros_like(w_ref)

    def x_copy(k, slot):
        b, r = k // _NCK, (k % _NCK) * _CR
        return pltpu.make_async_copy(
            x_hbm.at[b, pl.ds(r, _CR)], xbuf.at[slot], xsem.at[slot])

    def c_copy(k, slot):
        return pltpu.make_async_copy(
            c_hbm.at[pl.ds(k * _CCK, _CCK)], cbuf.at[slot], csem.at[slot])

    for k in range(_NBUF):
        x_copy(k, k).start()
    for k in range(_CBUF):
        c_copy(k, k).start()

    def x_body(k, carry):
        slot = k % _NBUF
        x_copy(k, slot).wait()
        x2 = xbuf[slot].reshape(_XR, _W)
        t1 = jnp.dot(p2_ref[k % 4], x2,
                     preferred_element_type=jnp.float32)
        sb = jnp.dot(t1, mw_ref[...],
                     preferred_element_type=jnp.float32)

        @pl.when(k + _NBUF < _NX)
        def _next():
            x_copy(k + _NBUF, slot).start()

        b = k // _NCK
        s_ref[pl.ds(_PH * b, _PH), :] += sb

        @pl.when(k < _NC)
        def _cstep():
            cslot = k % _CBUF
            c_copy(k, cslot).wait()
            wc = jnp.dot(pc_ref[...], cbuf[cslot],
                         preferred_element_type=jnp.float32)

            @pl.when(k + _CBUF < _NC)
            def _cnext():
                c_copy(k + _CBUF, cslot).start()

            w_ref[...] += wc

        return carry

    jax.lax.fori_loop(0, _NX, x_body, 0, unroll=False)

    w = w_ref[...].reshape(1, 1, _KOUT)
    for t in range(_B):
        o_ref[pl.ds(_PH * t, _PH)] = (
            s_ref[pl.ds(_PH * t, _PH), 0:_PW].reshape(_PH, _PW, 1) * w)


def kernel(x, C):
    xt = jnp.transpose(x, (0, 1, 3, 2))
    c3 = C.reshape(_CROWS, 128)
    mw = jnp.asarray(_MW)
    p2 = jnp.asarray(_P2)
    pc = jnp.asarray(_PC)
    out3 = pl.pallas_call(
        _reduce_kernel,
        in_specs=[
            pl.BlockSpec(memory_space=pltpu.HBM),
            pl.BlockSpec(memory_space=pltpu.HBM),
            pl.BlockSpec(memory_space=pltpu.MemorySpace.VMEM),
            pl.BlockSpec(memory_space=pltpu.MemorySpace.VMEM),
            pl.BlockSpec(memory_space=pltpu.MemorySpace.VMEM),
        ],
        out_specs=pl.BlockSpec(memory_space=pltpu.MemorySpace.VMEM),
        out_shape=jax.ShapeDtypeStruct((_B * _PH, _PW, _KOUT), jnp.float32),
        scratch_shapes=[
            pltpu.VMEM((_NBUF, _CR, _CIN, _W), jnp.float32),
            pltpu.VMEM((_CBUF, _CCK, 128), jnp.float32),
            pltpu.VMEM((_B * _PH, 128), jnp.float32),
            pltpu.VMEM((2, 128), jnp.float32),
            pltpu.SemaphoreType.DMA((_NBUF,)),
            pltpu.SemaphoreType.DMA((_CBUF,)),
        ],
    )(xt, c3, mw, p2, pc)
    out = out3.reshape(_B, _PH, _PW, _KOUT).transpose(0, 3, 1, 2)
    return out.reshape(_B, _KOUT, 1, _PH, _PW)
